# baseline (speedup 1.0000x reference)
import functools

import jax
import jax.numpy as jnp
from jax import lax
from jax.experimental import pallas as pl
from jax.experimental.pallas import tpu as pltpu
from jax.experimental.pallas import tpu_sc as plsc

BATCH = 16384
EMBED_DIM = 32
NUM_USERS = 1000000
NUM_MOVIES = 100000
NUM_CORES = 2
NUM_SUBCORES = 16
NUM_WORKERS = NUM_CORES * NUM_SUBCORES
BPW = BATCH // NUM_WORKERS
LANES = 16
WS = 512
NRING = 2
KCH = 4
GW = 8
PAD = 128
BSHIFT = 14
BMASK = (1 << BSHIFT) - 1
UNROLL = 4

UW = 61
U_TAIL_C0 = 1953 * WS
U_TAIL = NUM_USERS - U_TAIL_C0
MW = 6
M_TAIL_C0 = 195 * WS
M_TAIL = NUM_MOVIES - M_TAIL_C0

_mesh = plsc.VectorSubcoreMesh(
    core_axis_name="c", subcore_axis_name="s",
    num_cores=NUM_CORES, num_subcores=NUM_SUBCORES)

_NIDV = BATCH // LANES


def _popcnt(mask):
    return plsc.all_reduce_population_count(mask)[0]


@functools.partial(
    pl.kernel,
    out_type=(jax.ShapeDtypeStruct((BATCH + NUM_WORKERS * LANES, PAD),
                                   jnp.float32),
              jax.ShapeDtypeStruct((BATCH + NUM_WORKERS * LANES, PAD),
                                   jnp.float32)),
    mesh=_mesh,
    scratch_types=[
        pltpu.VMEM((BATCH + UNROLL * LANES,), jnp.int32),
        pltpu.VMEM((BATCH + UNROLL * LANES,), jnp.int32),
        pltpu.VMEM((BATCH + UNROLL * LANES,), jnp.int32),
        pltpu.VMEM((NRING * 4, 8, WS), jnp.float32),
        pltpu.VMEM((2 * KCH * LANES + LANES, PAD), jnp.float32),
        pltpu.VMEM((2 * KCH * LANES + LANES,), jnp.int32),
        pltpu.VMEM((U_TAIL, EMBED_DIM), jnp.float32),
        pltpu.VMEM((M_TAIL, EMBED_DIM), jnp.float32),
        pltpu.SemaphoreType.DMA,
        pltpu.SemaphoreType.DMA,
        pltpu.SemaphoreType.DMA,
    ],
    compiler_params=pltpu.CompilerParams(needs_layout_passes=False),
)
def _sc_scan(uids_hbm, mids_hbm, utab_hbm, mtab_hbm, utail_hbm, mtail_hbm,
             uemb_hbm, memb_hbm,
             wsel, sel, grp, slab, stag, bidx, utail_v, mtail_v,
             sem_ld, sem_st, sem_tl):
    wid = lax.axis_index("s") * NUM_CORES + lax.axis_index("c")
    last = wid == NUM_WORKERS - 1
    lanes = lax.iota(jnp.int32, LANES)
    dump = BATCH + wid * LANES + lanes

    NR = KCH * LANES

    def prefill(ring):
        for j in range(KCH):
            bidx[pl.ds(ring * NR + j * LANES, LANES)] = dump

    def st_issue(ring, emb_hbm):
        pltpu.async_copy(
            stag.at[pl.ds(ring * NR, NR)],
            emb_hbm.at[bidx.at[pl.ds(ring * NR, NR)]], sem_st)

    def st_drain():
        pltpu.make_async_copy(
            uemb_hbm.at[pl.ds(0, NR)],
            stag.at[pl.ds(0, NR)], sem_st).wait()

    def flush(ring, emb_hbm):
        st_drain()
        st_issue(ring, emb_hbm)
        ring = 1 - ring
        prefill(ring)
        return ring

    def select(ids_hbm, lo, hi):
        ids_v = wsel
        span_u = (hi - lo).astype(jnp.uint32)
        pltpu.sync_copy(ids_hbm, ids_v.at[pl.ds(0, BATCH)])

        def body(i, cnt):
            for j in range(UNROLL):
                k = i * UNROLL + j
                v = ids_v[pl.ds(k * LANES, LANES)]
                rel = v - lo
                m = rel.astype(jnp.uint32) < span_u
                pk = (rel << BSHIFT) | (k * LANES + lanes)
                plsc.store_compressed(sel.at[pl.ds(cnt, LANES)], pk, mask=m)
                cnt = cnt + _popcnt(m)
            return cnt

        cnt = lax.fori_loop(0, _NIDV // UNROLL, body, 0)
        for j in range(UNROLL):
            sel[pl.ds(cnt + j * LANES, LANES)] = jnp.full(
                (LANES,), 0x7FFFFFFF, jnp.int32)
        return cnt

    GSH = BSHIFT + 12

    def group_split(cnt):
        nblk = (cnt + UNROLL * LANES - 1) // (UNROLL * LANES)

        def count(i, cs):
            for j in range(UNROLL):
                pk = sel[pl.ds((i * UNROLL + j) * LANES, LANES)]
                g = pk >> GSH
                for q in range(8):
                    cs = cs[:q] + (cs[q] + _popcnt(g == q),) + cs[q + 1:]
            return cs

        counts = lax.fori_loop(0, nblk, count, (0,) * 8)
        offs = [0]
        for q in range(8):
            offs.append(offs[-1] + counts[q])

        def scat(i, cs):
            for j in range(UNROLL):
                pk = sel[pl.ds((i * UNROLL + j) * LANES, LANES)]
                g = pk >> GSH
                for q in range(8):
                    m = g == q
                    plsc.store_compressed(
                        grp.at[pl.ds(cs[q], LANES)], pk, mask=m)
                    cs = cs[:q] + (cs[q] + _popcnt(m),) + cs[q + 1:]
            return cs

        lax.fori_loop(0, nblk, scat, tuple(offs[:8]))
        grp[pl.ds(cnt, LANES)] = jnp.full((LANES,), 0x7FFFFFFF, jnp.int32)
        return offs

    def process(wrel0, span, sfrom, sto, gather_fn, emb_hbm, tot, pipelined):
        def wcomp(i, nw):
            for j in range(UNROLL):
                pk = grp[pl.ds(sfrom + (i * UNROLL + j) * LANES, LANES)]
                rel = pk >> BSHIFT
                m = (rel >= wrel0) & (rel < wrel0 + span)
                plsc.store_compressed(wsel.at[pl.ds(nw, LANES)], pk, mask=m)
                nw = nw + _popcnt(m)
            return nw

        nblk = (sto - sfrom + UNROLL * LANES - 1) // (UNROLL * LANES)
        nw = lax.fori_loop(0, nblk, wcomp, 0)

        def hchunk(h, fr):
            h0 = h * LANES
            pk = wsel[pl.ds(h0, LANES)]
            cols = (pk >> BSHIFT) - wrel0
            b = pk & BMASK
            valid = lanes < (nw - h0)
            cols = jnp.where(valid, cols, 0)
            bvec = jnp.where(valid, b, dump)
            if pipelined:
                f, ring = fr >> 1, fr & 1
                row0 = ring * NR + f * LANES
            else:
                row0 = 2 * NR
            for d in range(EMBED_DIM):
                vals = gather_fn(d, cols)
                plsc.store_scatter(
                    stag, [row0 + lanes,
                           jnp.full((LANES,), d, jnp.int32)], vals)
            if pipelined:
                bidx[pl.ds(row0, LANES)] = bvec
                f = f + 1

                def full_case():
                    return flush(ring, emb_hbm)

                def open_case():
                    return ring

                ring = lax.cond(f == KCH, full_case, open_case)
                f = jnp.where(f == KCH, 0, f)
                return (f << 1) | ring
            else:
                pltpu.async_copy(
                    stag.at[pl.ds(row0, LANES)], emb_hbm.at[bvec],
                    sem_tl).wait()
                return fr

        return lax.fori_loop(0, (nw + LANES - 1) // LANES, hchunk, tot)

    def ld_issue(tab_hbm, base_win, w, nfull):
        @pl.when(w < nfull)
        def _():
            p = lax.rem(w, NRING)
            pltpu.async_copy(
                tab_hbm.at[:, :, pl.ds((base_win + w) * WS, WS)],
                slab.at[pl.ds(p * 4, 4), :, :], sem_ld)

    def ld_drain(tab_hbm):
        pltpu.make_async_copy(
            tab_hbm.at[:, :, pl.ds(0, WS)],
            slab.at[pl.ds(0, 4), :, :], sem_ld).wait()

    def phase(ids_hbm, tab_hbm, emb_hbm, nfull, base_win, lo, hi, tot):
        cnt = select(ids_hbm, lo, hi)
        offs = group_split(cnt)
        for w in range(NRING - 1):
            ld_issue(tab_hbm, base_win, w, nfull)

        def slab_gather(p):
            def g(d, cols):
                return plsc.load_gather(
                    slab, [p * 4 + (d >> 3) + jnp.zeros((LANES,), jnp.int32),
                           jnp.full((LANES,), d & 7, jnp.int32), cols])
            return g

        for gq in range(8):
            nwin_g = jnp.clip(nfull - gq * GW, 0, GW)

            def wloop(i, tot, gq=gq):
                w = gq * GW + i
                p = lax.rem(w, NRING)
                ld_drain(tab_hbm)
                ld_issue(tab_hbm, base_win, w + NRING - 1, nfull)
                return process(w * WS + base_win * WS - lo, WS,
                               offs[gq], offs[gq + 1],
                               slab_gather(p), emb_hbm, tot, True)

            tot = lax.fori_loop(0, nwin_g, wloop, tot)
        return cnt, tot

    def tail(tail_hbm, tail_v, emb_hbm, tail_c0, span, lo, cnt):
        pltpu.sync_copy(tail_hbm, tail_v)

        def g(d, cols):
            return plsc.load_gather(
                tail_v, [cols, jnp.full((LANES,), d, jnp.int32)])

        process(tail_c0 - lo, span, 0, cnt, g, emb_hbm, 0, False)

    prefill(0)
    prefill(1)
    st_issue(1, uemb_hbm)

    u_nfull = jnp.where(last, UW + 1, UW)
    u_lo = wid * (UW * WS)
    u_hi = jnp.where(last, NUM_USERS, u_lo + UW * WS)
    ucnt, tot = phase(uids_hbm, utab_hbm, uemb_hbm, u_nfull, wid * UW,
                      u_lo, u_hi, 0)
    tot = lax.cond((tot & 1) == 1, lambda: flush(1, uemb_hbm),
                   lambda: flush(0, uemb_hbm))

    @pl.when(last)
    def _():
        tail(utail_hbm, utail_v, uemb_hbm, U_TAIL_C0, U_TAIL, u_lo, ucnt)

    m_nfull = jnp.where(last, MW + 3, MW)
    m_lo = wid * (MW * WS)
    m_hi = jnp.where(last, NUM_MOVIES, m_lo + MW * WS)
    mcnt, tot = phase(mids_hbm, mtab_hbm, memb_hbm, m_nfull, wid * MW,
                      m_lo, m_hi, tot)
    lax.cond((tot & 1) == 1, lambda: flush(1, memb_hbm),
             lambda: flush(0, memb_hbm))

    @pl.when(last)
    def _():
        tail(mtail_hbm, mtail_v, memb_hbm, M_TAIL_C0, M_TAIL, m_lo, mcnt)

    st_drain()


QROWS = BPW // 4


@functools.partial(
    pl.kernel,
    out_type=jax.ShapeDtypeStruct((BATCH,), jnp.float32),
    mesh=_mesh,
    scratch_types=[
        pltpu.VMEM((2 * QROWS, PAD), jnp.float32),
        pltpu.VMEM((2 * QROWS, PAD), jnp.float32),
        pltpu.VMEM((BPW,), jnp.float32),
        pltpu.SemaphoreType.DMA,
    ],
    compiler_params=pltpu.CompilerParams(needs_layout_passes=False),
)
def _sc_dot(uemb_hbm, memb_hbm, out_hbm, us, ms, outv, sem):
    wid = lax.axis_index("s") * NUM_CORES + lax.axis_index("c")
    base = wid * BPW
    lanes = lax.iota(jnp.int32, LANES)

    def issue(q):
        if q >= 4:
            return
        p = q & 1
        pltpu.async_copy(uemb_hbm.at[pl.ds(base + q * QROWS, QROWS), :],
                         us.at[pl.ds(p * QROWS, QROWS), :], sem)
        pltpu.async_copy(memb_hbm.at[pl.ds(base + q * QROWS, QROWS), :],
                         ms.at[pl.ds(p * QROWS, QROWS), :], sem)

    def drain():
        pltpu.make_async_copy(uemb_hbm.at[pl.ds(0, QROWS), :],
                              us.at[pl.ds(0, QROWS), :], sem).wait()
        pltpu.make_async_copy(memb_hbm.at[pl.ds(0, QROWS), :],
                              ms.at[pl.ds(0, QROWS), :], sem).wait()

    issue(0)
    for q in range(4):
        drain()
        issue(q + 1)
        p = q & 1

        def body(c, carry):
            ridx = (p * QROWS + c * LANES) + lanes
            acc = jnp.zeros((LANES,), jnp.float32)
            for d in range(EMBED_DIM):
                dv = jnp.full((LANES,), d, jnp.int32)
                acc = acc + (plsc.load_gather(us, [ridx, dv]) *
                             plsc.load_gather(ms, [ridx, dv]))
            outv[pl.ds(q * QROWS + c * LANES, LANES)] = acc
            return carry

        lax.fori_loop(0, QROWS // LANES, body, 0)

    pltpu.sync_copy(outv, out_hbm.at[pl.ds(base, BPW)])


def kernel(user_ids, movie_ids, user_table, movie_table):
    uemb, memb = _sc_scan(
        user_ids.astype(jnp.int32), movie_ids.astype(jnp.int32),
        user_table.T.reshape(4, 8, NUM_USERS),
        movie_table.T.reshape(4, 8, NUM_MOVIES),
        user_table[U_TAIL_C0:], movie_table[M_TAIL_C0:])
    return _sc_dot(uemb, memb)


# speedup vs baseline: 1.2214x; 1.2214x over previous
import functools

import jax
import jax.numpy as jnp
from jax import lax
from jax.experimental import pallas as pl
from jax.experimental.pallas import tpu as pltpu
from jax.experimental.pallas import tpu_sc as plsc

BATCH = 16384
EMBED_DIM = 32
NUM_USERS = 1000000
NUM_MOVIES = 100000
NUM_CORES = 2
NUM_SUBCORES = 16
NUM_WORKERS = NUM_CORES * NUM_SUBCORES
BPW = BATCH // NUM_WORKERS
LANES = 16
WS = 512
NRING = 3
KCH = 4
PAD = 128
BSHIFT = 14
BMASK = (1 << BSHIFT) - 1
UNROLL = 4

UW = 61
U_TAIL_C0 = 1953 * WS
U_TAIL = NUM_USERS - U_TAIL_C0
MW = 6
M_TAIL_C0 = 195 * WS
M_TAIL = NUM_MOVIES - M_TAIL_C0

_mesh = plsc.VectorSubcoreMesh(
    core_axis_name="c", subcore_axis_name="s",
    num_cores=NUM_CORES, num_subcores=NUM_SUBCORES)

_NIDV = BATCH // LANES


def _popcnt(mask):
    return plsc.all_reduce_population_count(mask)[0]


@functools.partial(
    pl.kernel,
    out_type=(jax.ShapeDtypeStruct((BATCH + NUM_WORKERS * LANES, PAD),
                                   jnp.float32),
              jax.ShapeDtypeStruct((BATCH + NUM_WORKERS * LANES, PAD),
                                   jnp.float32)),
    mesh=_mesh,
    scratch_types=[
        pltpu.VMEM((BATCH + UNROLL * LANES,), jnp.int32),
        pltpu.VMEM((BATCH + UNROLL * LANES,), jnp.int32),
        pltpu.VMEM((NRING * 4, 8, WS), jnp.float32),
        pltpu.VMEM((2 * KCH * LANES + LANES, PAD), jnp.float32),
        pltpu.VMEM((2 * KCH * LANES + LANES,), jnp.int32),
        pltpu.VMEM((U_TAIL, EMBED_DIM), jnp.float32),
        pltpu.VMEM((M_TAIL, EMBED_DIM), jnp.float32),
        pltpu.SemaphoreType.DMA,
        pltpu.SemaphoreType.DMA,
        pltpu.SemaphoreType.DMA,
    ],
    compiler_params=pltpu.CompilerParams(needs_layout_passes=False),
)
def _sc_scan(uids_hbm, mids_hbm, utab_hbm, mtab_hbm, utail_hbm, mtail_hbm,
             uemb_hbm, memb_hbm,
             wsel, sel, slab, stag, bidx, utail_v, mtail_v,
             sem_ld, sem_st, sem_tl):
    wid = lax.axis_index("s") * NUM_CORES + lax.axis_index("c")
    last = wid == NUM_WORKERS - 1
    lanes = lax.iota(jnp.int32, LANES)
    dump = BATCH + wid * LANES + lanes

    NR = KCH * LANES

    def prefill(ring):
        for j in range(KCH):
            bidx[pl.ds(ring * NR + j * LANES, LANES)] = dump

    def st_issue(ring, emb_hbm):
        pltpu.async_copy(
            stag.at[pl.ds(ring * NR, NR)],
            emb_hbm.at[bidx.at[pl.ds(ring * NR, NR)]], sem_st)

    def st_drain():
        pltpu.make_async_copy(
            uemb_hbm.at[pl.ds(0, NR)],
            stag.at[pl.ds(0, NR)], sem_st).wait()

    def flush(ring, emb_hbm):
        st_drain()
        st_issue(ring, emb_hbm)
        ring = 1 - ring
        prefill(ring)
        return ring

    def select(ids_hbm, lo, hi):
        ids_v = wsel
        span_u = (hi - lo).astype(jnp.uint32)
        pltpu.sync_copy(ids_hbm, ids_v.at[pl.ds(0, BATCH)])

        def body(i, cnt):
            for j in range(UNROLL):
                k = i * UNROLL + j
                v = ids_v[pl.ds(k * LANES, LANES)]
                rel = v - lo
                m = rel.astype(jnp.uint32) < span_u
                pk = (rel << BSHIFT) | (k * LANES + lanes)
                plsc.store_compressed(sel.at[pl.ds(cnt, LANES)], pk, mask=m)
                cnt = cnt + _popcnt(m)
            return cnt

        cnt = lax.fori_loop(0, _NIDV // UNROLL, body, 0)
        for j in range(UNROLL):
            sel[pl.ds(cnt + j * LANES, LANES)] = jnp.full(
                (LANES,), 0x7FFFFFFF, jnp.int32)
        return cnt

    def process(wrel0, span, cnt, gather_fn, emb_hbm, tot, pipelined):
        def wcomp(i, nw):
            for j in range(UNROLL):
                pk = sel[pl.ds((i * UNROLL + j) * LANES, LANES)]
                rel = pk >> BSHIFT
                m = (rel >= wrel0) & (rel < wrel0 + span)
                plsc.store_compressed(wsel.at[pl.ds(nw, LANES)], pk, mask=m)
                nw = nw + _popcnt(m)
            return nw

        nblk = (cnt + UNROLL * LANES - 1) // (UNROLL * LANES)
        nw = lax.fori_loop(0, nblk, wcomp, 0)

        def hchunk(h, fr):
            h0 = h * LANES
            pk = wsel[pl.ds(h0, LANES)]
            cols = (pk >> BSHIFT) - wrel0
            b = pk & BMASK
            valid = lanes < (nw - h0)
            cols = jnp.where(valid, cols, 0)
            bvec = jnp.where(valid, b, dump)
            if pipelined:
                f, ring = fr >> 1, fr & 1
                row0 = ring * NR + f * LANES
            else:
                row0 = 2 * NR
            for d in range(EMBED_DIM):
                vals = gather_fn(d, cols)
                plsc.store_scatter(
                    stag, [row0 + lanes,
                           jnp.full((LANES,), d, jnp.int32)], vals)
            if pipelined:
                bidx[pl.ds(row0, LANES)] = bvec
                f = f + 1

                def full_case():
                    return flush(ring, emb_hbm)

                def open_case():
                    return ring

                ring = lax.cond(f == KCH, full_case, open_case)
                f = jnp.where(f == KCH, 0, f)
                return (f << 1) | ring
            else:
                pltpu.async_copy(
                    stag.at[pl.ds(row0, LANES)], emb_hbm.at[bvec],
                    sem_tl).wait()
                return fr

        return lax.fori_loop(0, (nw + LANES - 1) // LANES, hchunk, tot)

    def ld_issue(tab_hbm, base_win, w, nfull):
        @pl.when(w < nfull)
        def _():
            p = lax.rem(w, NRING)
            pltpu.async_copy(
                tab_hbm.at[:, :, pl.ds((base_win + w) * WS, WS)],
                slab.at[pl.ds(p * 4, 4), :, :], sem_ld)

    def ld_drain(tab_hbm):
        pltpu.make_async_copy(
            tab_hbm.at[:, :, pl.ds(0, WS)],
            slab.at[pl.ds(0, 4), :, :], sem_ld).wait()

    def phase(ids_hbm, tab_hbm, emb_hbm, nfull, base_win, lo, hi, tot):
        cnt = select(ids_hbm, lo, hi)
        for w in range(NRING - 1):
            ld_issue(tab_hbm, base_win, w, nfull)

        def slab_gather(p):
            def g(d, cols):
                return plsc.load_gather(
                    slab, [p * 4 + (d >> 3) + jnp.zeros((LANES,), jnp.int32),
                           jnp.full((LANES,), d & 7, jnp.int32), cols])
            return g

        def wloop(w, tot):
            p = lax.rem(w, NRING)
            ld_drain(tab_hbm)
            ld_issue(tab_hbm, base_win, w + NRING - 1, nfull)
            return process(w * WS + base_win * WS - lo, WS, cnt,
                           slab_gather(p), emb_hbm, tot, True)

        tot = lax.fori_loop(0, nfull, wloop, tot)
        return cnt, tot

    def tail(tail_hbm, tail_v, emb_hbm, tail_c0, span, lo, cnt):
        pltpu.sync_copy(tail_hbm, tail_v)

        def g(d, cols):
            return plsc.load_gather(
                tail_v, [cols, jnp.full((LANES,), d, jnp.int32)])

        process(tail_c0 - lo, span, cnt, g, emb_hbm, 0, False)

    prefill(0)
    prefill(1)
    st_issue(1, uemb_hbm)

    u_nfull = jnp.where(last, UW + 1, UW)
    u_lo = wid * (UW * WS)
    u_hi = jnp.where(last, NUM_USERS, u_lo + UW * WS)
    ucnt, tot = phase(uids_hbm, utab_hbm, uemb_hbm, u_nfull, wid * UW,
                      u_lo, u_hi, 0)
    tot = lax.cond((tot & 1) == 1, lambda: flush(1, uemb_hbm),
                   lambda: flush(0, uemb_hbm))

    @pl.when(last)
    def _():
        tail(utail_hbm, utail_v, uemb_hbm, U_TAIL_C0, U_TAIL, u_lo, ucnt)

    m_nfull = jnp.where(last, MW + 3, MW)
    m_lo = wid * (MW * WS)
    m_hi = jnp.where(last, NUM_MOVIES, m_lo + MW * WS)
    mcnt, tot = phase(mids_hbm, mtab_hbm, memb_hbm, m_nfull, wid * MW,
                      m_lo, m_hi, tot)
    lax.cond((tot & 1) == 1, lambda: flush(1, memb_hbm),
             lambda: flush(0, memb_hbm))

    @pl.when(last)
    def _():
        tail(mtail_hbm, mtail_v, memb_hbm, M_TAIL_C0, M_TAIL, m_lo, mcnt)

    st_drain()


QROWS = BPW // 4


@functools.partial(
    pl.kernel,
    out_type=jax.ShapeDtypeStruct((BATCH,), jnp.float32),
    mesh=_mesh,
    scratch_types=[
        pltpu.VMEM((2 * QROWS, PAD), jnp.float32),
        pltpu.VMEM((2 * QROWS, PAD), jnp.float32),
        pltpu.VMEM((BPW,), jnp.float32),
        pltpu.SemaphoreType.DMA,
    ],
    compiler_params=pltpu.CompilerParams(needs_layout_passes=False),
)
def _sc_dot(uemb_hbm, memb_hbm, out_hbm, us, ms, outv, sem):
    wid = lax.axis_index("s") * NUM_CORES + lax.axis_index("c")
    base = wid * BPW
    lanes = lax.iota(jnp.int32, LANES)

    def issue(q):
        if q >= 4:
            return
        p = q & 1
        pltpu.async_copy(uemb_hbm.at[pl.ds(base + q * QROWS, QROWS), :],
                         us.at[pl.ds(p * QROWS, QROWS), :], sem)
        pltpu.async_copy(memb_hbm.at[pl.ds(base + q * QROWS, QROWS), :],
                         ms.at[pl.ds(p * QROWS, QROWS), :], sem)

    def drain():
        pltpu.make_async_copy(uemb_hbm.at[pl.ds(0, QROWS), :],
                              us.at[pl.ds(0, QROWS), :], sem).wait()
        pltpu.make_async_copy(memb_hbm.at[pl.ds(0, QROWS), :],
                              ms.at[pl.ds(0, QROWS), :], sem).wait()

    issue(0)
    for q in range(4):
        drain()
        issue(q + 1)
        p = q & 1

        def body(c, carry):
            ridx = (p * QROWS + c * LANES) + lanes
            acc = jnp.zeros((LANES,), jnp.float32)
            for d in range(EMBED_DIM):
                dv = jnp.full((LANES,), d, jnp.int32)
                acc = acc + (plsc.load_gather(us, [ridx, dv]) *
                             plsc.load_gather(ms, [ridx, dv]))
            outv[pl.ds(q * QROWS + c * LANES, LANES)] = acc
            return carry

        lax.fori_loop(0, QROWS // LANES, body, 0)

    pltpu.sync_copy(outv, out_hbm.at[pl.ds(base, BPW)])


def kernel(user_ids, movie_ids, user_table, movie_table):
    uemb, memb = _sc_scan(
        user_ids.astype(jnp.int32), movie_ids.astype(jnp.int32),
        user_table.T.reshape(4, 8, NUM_USERS),
        movie_table.T.reshape(4, 8, NUM_MOVIES),
        user_table[U_TAIL_C0:], movie_table[M_TAIL_C0:])
    return _sc_dot(uemb, memb)
